# baseline (speedup 1.0000x reference)
import jax
import jax.numpy as jnp
from jax import lax
from jax.experimental import pallas as pl
from jax.experimental.pallas import tpu as pltpu
from jax.experimental.pallas import tpu_sc as plsc

NUM_WORKERS = 32
ROWS_PER_WORKER = 512
CHUNK_SIZES = (32, 32, 64, 128, 128, 64, 32, 32)
CHUNK_OFFS = (0, 32, 64, 128, 256, 384, 448, 480)


def _gather_kernel(idx_hbm, table_hbm, out_hbm, idx_v, rows_v, tbl_sh, *sems):
    sid = lax.axis_index("s")
    wid = sid * 2 + lax.axis_index("c")
    base = wid * ROWS_PER_WORKER
    sem_o = sems[len(CHUNK_SIZES)]
    idx_cp = pltpu.async_copy(idx_hbm.at[wid], idx_v, sems[0])

    @pl.when(sid == 0)
    def _stage_table():
        pltpu.sync_copy(table_hbm, tbl_sh)

    plsc.subcore_barrier()
    idx_cp.wait()

    gathers = [
        pltpu.async_copy(tbl_sh.at[idx_v.at[pl.ds(off, n)]],
                         rows_v.at[pl.ds(off, n)], sems[j])
        for j, (off, n) in enumerate(zip(CHUNK_OFFS, CHUNK_SIZES))
    ]
    writes = []
    for j, (off, n) in enumerate(zip(CHUNK_OFFS, CHUNK_SIZES)):
        gathers[j].wait()
        writes.append(
            pltpu.async_copy(rows_v.at[pl.ds(off, n)],
                             out_hbm.at[pl.ds(base + off, n)], sem_o))
    for c in writes:
        c.wait()


def kernel(indices, table):
    batch = indices.shape[0]
    keys, dims = table.shape
    idx2d = indices.astype(jnp.int32).reshape(NUM_WORKERS, ROWS_PER_WORKER)
    mesh = plsc.VectorSubcoreMesh(core_axis_name="c", subcore_axis_name="s")
    out = pl.kernel(
        _gather_kernel,
        out_type=jax.ShapeDtypeStruct((batch, dims), jnp.float32),
        mesh=mesh,
        scratch_types=[
            pltpu.VMEM((ROWS_PER_WORKER,), jnp.int32),
            pltpu.VMEM((ROWS_PER_WORKER, dims), jnp.float32),
            pltpu.VMEM_SHARED((keys, dims), jnp.float32),
        ] + [pltpu.SemaphoreType.DMA] * (len(CHUNK_SIZES) + 1),
    )(idx2d, table)
    return out
